# Initial kernel scaffold; baseline (speedup 1.0000x reference)
import jax
import jax.numpy as jnp
from jax.experimental import pallas as pl


def kernel(x, W_gate):
    raise NotImplementedError("write your pallas kernel here")


# speedup vs baseline: 1.8044x; 1.8044x over previous
import functools

import jax
import jax.numpy as jnp
from jax.experimental import pallas as pl

_NUM_EXPERTS = 64
_TOP_K = 2
_NOISE_STD = 1e-2
_BT = 512


def _router_block(x_ref, w_ref, noise_ref, combine_ref, idx_ref, score_ref):
    logits = jnp.dot(x_ref[...], w_ref[...], preferred_element_type=jnp.float32)
    noisy = logits + noise_ref[...]

    eidx = jax.lax.broadcasted_iota(jnp.int32, noisy.shape, 1)
    m1 = jnp.max(noisy, axis=-1, keepdims=True)
    a1 = jnp.min(jnp.where(noisy == m1, eidx, _NUM_EXPERTS), axis=-1, keepdims=True)
    masked = jnp.where(eidx == a1, -jnp.inf, noisy)
    m2 = jnp.max(masked, axis=-1, keepdims=True)
    a2 = jnp.min(jnp.where(masked == m2, eidx, _NUM_EXPERTS), axis=-1, keepdims=True)

    z = jnp.sum(jnp.exp(noisy - m1), axis=-1, keepdims=True)
    p1 = 1.0 / z
    p2 = jnp.exp(m2 - m1) / z
    denom = p1 + p2 + 1e-9
    w1 = p1 / denom
    w2 = p2 / denom

    combine_ref[...] = (
        jnp.where(eidx == a1, w1, 0.0) + jnp.where(eidx == a2, w2, 0.0)
    )
    idx_ref[...] = jnp.concatenate([a1, a2], axis=-1)
    score_ref[...] = jnp.concatenate([w1, w2], axis=-1)


@jax.jit
def kernel(x, W_gate):
    tokens = x.shape[0]
    noise = (
        jax.random.normal(jax.random.key(42), (tokens, _NUM_EXPERTS), dtype=jnp.float32)
        * _NOISE_STD
    )
    grid = (tokens // _BT,)
    combine, indices, scores = pl.pallas_call(
        _router_block,
        grid=grid,
        in_specs=[
            pl.BlockSpec((_BT, x.shape[1]), lambda i: (i, 0)),
            pl.BlockSpec((x.shape[1], _NUM_EXPERTS), lambda i: (0, 0)),
            pl.BlockSpec((_BT, _NUM_EXPERTS), lambda i: (i, 0)),
        ],
        out_specs=[
            pl.BlockSpec((_BT, _NUM_EXPERTS), lambda i: (i, 0)),
            pl.BlockSpec((_BT, _TOP_K), lambda i: (i, 0)),
            pl.BlockSpec((_BT, _TOP_K), lambda i: (i, 0)),
        ],
        out_shape=[
            jax.ShapeDtypeStruct((tokens, _NUM_EXPERTS), jnp.float32),
            jax.ShapeDtypeStruct((tokens, _TOP_K), jnp.int32),
            jax.ShapeDtypeStruct((tokens, _TOP_K), jnp.float32),
        ],
    )(x, W_gate, noise)
    return combine, indices, scores


# speedup vs baseline: 1.8050x; 1.0003x over previous
import functools

import jax
import jax.numpy as jnp
from jax.experimental import pallas as pl

_NUM_EXPERTS = 64
_TOP_K = 2
_NOISE_STD = 1e-2
_BT = 512


def _router_block(x_ref, w_ref, noise_ref, combine_ref, idx_ref, score_ref):
    logits = jnp.dot(x_ref[...], w_ref[...], preferred_element_type=jnp.float32)
    noisy = logits + noise_ref[...]

    eidx = jax.lax.broadcasted_iota(jnp.int32, noisy.shape, 1)
    m1 = jnp.max(noisy, axis=-1, keepdims=True)
    a1 = jnp.min(jnp.where(noisy == m1, eidx, _NUM_EXPERTS), axis=-1, keepdims=True)
    masked = jnp.where(eidx == a1, -jnp.inf, noisy)
    m2 = jnp.max(masked, axis=-1, keepdims=True)
    a2 = jnp.min(jnp.where(masked == m2, eidx, _NUM_EXPERTS), axis=-1, keepdims=True)

    z = jnp.sum(jnp.exp(noisy - m1), axis=-1, keepdims=True)
    p1 = 1.0 / z
    p2 = jnp.exp(m2 - m1) / z
    denom = p1 + p2 + 1e-9
    w1 = p1 / denom
    w2 = p2 / denom

    combine_ref[...] = (
        jnp.where(eidx == a1, w1, 0.0) + jnp.where(eidx == a2, w2, 0.0)
    )
    idx_ref[...] = jnp.concatenate([a1, a2], axis=-1)
    score_ref[...] = jnp.concatenate([w1, w2], axis=-1)


@functools.lru_cache(maxsize=2)
def _noise_const(tokens):
    return jax.random.normal(
        jax.random.key(42), (tokens, _NUM_EXPERTS), dtype=jnp.float32
    ) * _NOISE_STD


@jax.jit
def kernel(x, W_gate):
    tokens = x.shape[0]
    noise = _noise_const(tokens)
    grid = (tokens // _BT,)
    combine, indices, scores = pl.pallas_call(
        _router_block,
        grid=grid,
        in_specs=[
            pl.BlockSpec((_BT, x.shape[1]), lambda i: (i, 0)),
            pl.BlockSpec((x.shape[1], _NUM_EXPERTS), lambda i: (0, 0)),
            pl.BlockSpec((_BT, _NUM_EXPERTS), lambda i: (i, 0)),
        ],
        out_specs=[
            pl.BlockSpec((_BT, _NUM_EXPERTS), lambda i: (i, 0)),
            pl.BlockSpec((_BT, _TOP_K), lambda i: (i, 0)),
            pl.BlockSpec((_BT, _TOP_K), lambda i: (i, 0)),
        ],
        out_shape=[
            jax.ShapeDtypeStruct((tokens, _NUM_EXPERTS), jnp.float32),
            jax.ShapeDtypeStruct((tokens, _TOP_K), jnp.int32),
            jax.ShapeDtypeStruct((tokens, _TOP_K), jnp.float32),
        ],
    )(x, W_gate, noise)
    return combine, indices, scores


# speedup vs baseline: 3.1603x; 1.7509x over previous
import functools

import jax
import jax.numpy as jnp
from jax.experimental import pallas as pl

_NUM_EXPERTS = 64
_TOP_K = 2
_NOISE_STD = 1e-2
_BT = 512


def _router_block(x_ref, w_ref, noise_ref, combine_ref, idx_ref, score_ref):
    logits = jnp.dot(x_ref[...], w_ref[...], preferred_element_type=jnp.float32)
    noisy = logits + noise_ref[...]

    eidx = jax.lax.broadcasted_iota(jnp.int32, noisy.shape, 1)
    m1 = jnp.max(noisy, axis=-1, keepdims=True)
    a1 = jnp.min(jnp.where(noisy == m1, eidx, _NUM_EXPERTS), axis=-1, keepdims=True)
    masked = jnp.where(eidx == a1, -jnp.inf, noisy)
    m2 = jnp.max(masked, axis=-1, keepdims=True)
    a2 = jnp.min(jnp.where(masked == m2, eidx, _NUM_EXPERTS), axis=-1, keepdims=True)

    z = jnp.sum(jnp.exp(noisy - m1), axis=-1, keepdims=True)
    p1 = 1.0 / z
    p2 = jnp.exp(m2 - m1) / z
    denom = p1 + p2 + 1e-9
    w1 = p1 / denom
    w2 = p2 / denom

    combine_ref[...] = (
        jnp.where(eidx == a1, w1, 0.0) + jnp.where(eidx == a2, w2, 0.0)
    )
    idx_ref[...] = jnp.concatenate([a1, a2], axis=-1)
    score_ref[...] = jnp.concatenate([w1, w2], axis=-1)


@functools.lru_cache(maxsize=2)
def _noise_const(tokens):
    with jax.ensure_compile_time_eval():
        return jax.random.normal(
            jax.random.key(42), (tokens, _NUM_EXPERTS), dtype=jnp.float32
        ) * _NOISE_STD


@jax.jit
def kernel(x, W_gate):
    tokens = x.shape[0]
    noise = _noise_const(tokens)
    grid = (tokens // _BT,)
    combine, indices, scores = pl.pallas_call(
        _router_block,
        grid=grid,
        in_specs=[
            pl.BlockSpec((_BT, x.shape[1]), lambda i: (i, 0)),
            pl.BlockSpec((x.shape[1], _NUM_EXPERTS), lambda i: (0, 0)),
            pl.BlockSpec((_BT, _NUM_EXPERTS), lambda i: (i, 0)),
        ],
        out_specs=[
            pl.BlockSpec((_BT, _NUM_EXPERTS), lambda i: (i, 0)),
            pl.BlockSpec((_BT, _TOP_K), lambda i: (i, 0)),
            pl.BlockSpec((_BT, _TOP_K), lambda i: (i, 0)),
        ],
        out_shape=[
            jax.ShapeDtypeStruct((tokens, _NUM_EXPERTS), jnp.float32),
            jax.ShapeDtypeStruct((tokens, _TOP_K), jnp.int32),
            jax.ShapeDtypeStruct((tokens, _TOP_K), jnp.float32),
        ],
    )(x, W_gate, noise)
    return combine, indices, scores


# speedup vs baseline: 5.5992x; 1.7717x over previous
import functools

import jax
import jax.numpy as jnp
from jax.experimental import pallas as pl

_NUM_EXPERTS = 64
_TOP_K = 2
_NOISE_STD = 1e-2
_BT = 512


def _router_block(x_ref, w_ref, noise_ref, combine_ref, idx_ref, score_ref):
    logits = jnp.dot(x_ref[...], w_ref[...], preferred_element_type=jnp.float32)
    noisy = logits.T + noise_ref[...]

    eidx = jax.lax.broadcasted_iota(jnp.int32, noisy.shape, 0)
    m1 = jnp.max(noisy, axis=0, keepdims=True)
    a1 = jnp.min(jnp.where(noisy == m1, eidx, _NUM_EXPERTS), axis=0, keepdims=True)
    masked = jnp.where(eidx == a1, -jnp.inf, noisy)
    m2 = jnp.max(masked, axis=0, keepdims=True)
    a2 = jnp.min(jnp.where(masked == m2, eidx, _NUM_EXPERTS), axis=0, keepdims=True)

    z = jnp.sum(jnp.exp(noisy - m1), axis=0, keepdims=True)
    p1 = 1.0 / z
    p2 = jnp.exp(m2 - m1) / z
    denom = p1 + p2 + 1e-9
    w1 = p1 / denom
    w2 = p2 / denom

    combine_ref[...] = (
        jnp.where(eidx == a1, w1, 0.0) + jnp.where(eidx == a2, w2, 0.0)
    )
    idx_ref[...] = jnp.concatenate([a1, a2], axis=0)
    score_ref[...] = jnp.concatenate([w1, w2], axis=0)


@functools.lru_cache(maxsize=2)
def _noise_const(tokens):
    def _gen():
        return (
            jax.random.normal(
                jax.random.key(42), (tokens, _NUM_EXPERTS), dtype=jnp.float32
            )
            * _NOISE_STD
        ).T

    try:
        with jax.ensure_compile_time_eval():
            return _gen()
    except Exception:
        return _gen()


@jax.jit
def kernel(x, W_gate):
    tokens = x.shape[0]
    noise_t = _noise_const(tokens)
    grid = (tokens // _BT,)
    combine_t, indices_t, scores_t = pl.pallas_call(
        _router_block,
        grid=grid,
        in_specs=[
            pl.BlockSpec((_BT, x.shape[1]), lambda i: (i, 0)),
            pl.BlockSpec((x.shape[1], _NUM_EXPERTS), lambda i: (0, 0)),
            pl.BlockSpec((_NUM_EXPERTS, _BT), lambda i: (0, i)),
        ],
        out_specs=[
            pl.BlockSpec((_NUM_EXPERTS, _BT), lambda i: (0, i)),
            pl.BlockSpec((_TOP_K, _BT), lambda i: (0, i)),
            pl.BlockSpec((_TOP_K, _BT), lambda i: (0, i)),
        ],
        out_shape=[
            jax.ShapeDtypeStruct((_NUM_EXPERTS, tokens), jnp.float32),
            jax.ShapeDtypeStruct((_TOP_K, tokens), jnp.int32),
            jax.ShapeDtypeStruct((_TOP_K, tokens), jnp.float32),
        ],
    )(x, W_gate, noise_t)
    return combine_t.T, indices_t.T, scores_t.T


# speedup vs baseline: 7.9077x; 1.4123x over previous
import functools

import jax
import jax.numpy as jnp
from jax.experimental import pallas as pl

_NUM_EXPERTS = 64
_TOP_K = 2
_NOISE_STD = 1e-2
_BT = 1024


def _router_block(x_ref, w_ref, noise_ref, combine_ref, idx_ref, score_ref):
    logits = jnp.dot(x_ref[...], w_ref[...], preferred_element_type=jnp.float32)
    noisy = logits.T + noise_ref[...]

    eidx = jax.lax.broadcasted_iota(jnp.int32, noisy.shape, 0)
    m1 = jnp.max(noisy, axis=0, keepdims=True)
    a1 = jnp.min(jnp.where(noisy == m1, eidx, _NUM_EXPERTS), axis=0, keepdims=True)
    masked = jnp.where(eidx == a1, -jnp.inf, noisy)
    m2 = jnp.max(masked, axis=0, keepdims=True)
    a2 = jnp.min(jnp.where(masked == m2, eidx, _NUM_EXPERTS), axis=0, keepdims=True)

    z = jnp.sum(jnp.exp(noisy - m1), axis=0, keepdims=True)
    p1 = 1.0 / z
    p2 = jnp.exp(m2 - m1) / z
    denom = p1 + p2 + 1e-9
    w1 = p1 / denom
    w2 = p2 / denom

    combine_ref[...] = (
        jnp.where(eidx == a1, w1, 0.0) + jnp.where(eidx == a2, w2, 0.0)
    )
    idx_ref[...] = jnp.concatenate([a1, a2], axis=0)
    score_ref[...] = jnp.concatenate([w1, w2], axis=0)


@functools.lru_cache(maxsize=2)
def _noise_const(tokens):
    def _gen():
        return (
            jax.random.normal(
                jax.random.key(42), (tokens, _NUM_EXPERTS), dtype=jnp.float32
            )
            * _NOISE_STD
        ).T

    try:
        with jax.ensure_compile_time_eval():
            return _gen()
    except Exception:
        return _gen()


@jax.jit
def kernel(x, W_gate):
    tokens = x.shape[0]
    noise_t = _noise_const(tokens)
    grid = (tokens // _BT,)
    combine_t, indices_t, scores_t = pl.pallas_call(
        _router_block,
        grid=grid,
        in_specs=[
            pl.BlockSpec((_BT, x.shape[1]), lambda i: (i, 0)),
            pl.BlockSpec((x.shape[1], _NUM_EXPERTS), lambda i: (0, 0)),
            pl.BlockSpec((_NUM_EXPERTS, _BT), lambda i: (0, i)),
        ],
        out_specs=[
            pl.BlockSpec((_NUM_EXPERTS, _BT), lambda i: (0, i)),
            pl.BlockSpec((_TOP_K, _BT), lambda i: (0, i)),
            pl.BlockSpec((_TOP_K, _BT), lambda i: (0, i)),
        ],
        out_shape=[
            jax.ShapeDtypeStruct((_NUM_EXPERTS, tokens), jnp.float32),
            jax.ShapeDtypeStruct((_TOP_K, tokens), jnp.int32),
            jax.ShapeDtypeStruct((_TOP_K, tokens), jnp.float32),
        ],
    )(x, W_gate, noise_t)
    return combine_t.T, indices_t.T, scores_t.T


# speedup vs baseline: 9.2833x; 1.1740x over previous
import functools

import jax
import jax.numpy as jnp
from jax.experimental import pallas as pl

_NUM_EXPERTS = 64
_TOP_K = 2
_NOISE_STD = 1e-2
_BT = 2048


def _router_block(x_ref, w_ref, noise_ref, combine_ref, idx_ref, score_ref):
    logits = jnp.dot(x_ref[...], w_ref[...], preferred_element_type=jnp.float32)
    noisy = logits.T + noise_ref[...]

    eidx = jax.lax.broadcasted_iota(jnp.int32, noisy.shape, 0)
    m1 = jnp.max(noisy, axis=0, keepdims=True)
    a1 = jnp.min(jnp.where(noisy == m1, eidx, _NUM_EXPERTS), axis=0, keepdims=True)
    masked = jnp.where(eidx == a1, -jnp.inf, noisy)
    m2 = jnp.max(masked, axis=0, keepdims=True)
    a2 = jnp.min(jnp.where(masked == m2, eidx, _NUM_EXPERTS), axis=0, keepdims=True)

    z = jnp.sum(jnp.exp(noisy - m1), axis=0, keepdims=True)
    p1 = 1.0 / z
    p2 = jnp.exp(m2 - m1) / z
    denom = p1 + p2 + 1e-9
    w1 = p1 / denom
    w2 = p2 / denom

    combine_ref[...] = (
        jnp.where(eidx == a1, w1, 0.0) + jnp.where(eidx == a2, w2, 0.0)
    )
    idx_ref[...] = jnp.concatenate([a1, a2], axis=0)
    score_ref[...] = jnp.concatenate([w1, w2], axis=0)


@functools.lru_cache(maxsize=2)
def _noise_const(tokens):
    def _gen():
        return (
            jax.random.normal(
                jax.random.key(42), (tokens, _NUM_EXPERTS), dtype=jnp.float32
            )
            * _NOISE_STD
        ).T

    try:
        with jax.ensure_compile_time_eval():
            return _gen()
    except Exception:
        return _gen()


@jax.jit
def kernel(x, W_gate):
    tokens = x.shape[0]
    noise_t = _noise_const(tokens)
    grid = (tokens // _BT,)
    combine_t, indices_t, scores_t = pl.pallas_call(
        _router_block,
        grid=grid,
        in_specs=[
            pl.BlockSpec((_BT, x.shape[1]), lambda i: (i, 0)),
            pl.BlockSpec((x.shape[1], _NUM_EXPERTS), lambda i: (0, 0)),
            pl.BlockSpec((_NUM_EXPERTS, _BT), lambda i: (0, i)),
        ],
        out_specs=[
            pl.BlockSpec((_NUM_EXPERTS, _BT), lambda i: (0, i)),
            pl.BlockSpec((_TOP_K, _BT), lambda i: (0, i)),
            pl.BlockSpec((_TOP_K, _BT), lambda i: (0, i)),
        ],
        out_shape=[
            jax.ShapeDtypeStruct((_NUM_EXPERTS, tokens), jnp.float32),
            jax.ShapeDtypeStruct((_TOP_K, tokens), jnp.int32),
            jax.ShapeDtypeStruct((_TOP_K, tokens), jnp.float32),
        ],
    )(x, W_gate, noise_t)
    return combine_t.T, indices_t.T, scores_t.T
